# Initial kernel scaffold; baseline (speedup 1.0000x reference)
import jax
import jax.numpy as jnp
from jax.experimental import pallas as pl


def kernel(h, edge_index, etype, norm, weight, h_bias):
    raise NotImplementedError("write your pallas kernel here")


# speedup vs baseline: 17.3143x; 17.3143x over previous
import functools

import jax
import jax.numpy as jnp
from jax import lax
from jax.experimental import pallas as pl
from jax.experimental.pallas import tpu as pltpu
from jax.experimental.pallas import tpu_sc as plsc

N_NODES = 10000
N_EDGES = 320000
IN_FEAT = 128
OUT_FEAT = 128
NUM_RELS = 64
NUM_BASES = 8
SI = IN_FEAT // NUM_BASES
SO = OUT_FEAT // NUM_BASES

NC = 2
NS = 16
NW = NC * NS

EDGES_PER_TILE = N_EDGES // NW
CHUNK = 80
N_CHUNKS = EDGES_PER_TILE // CHUNK
ROWS_BIG = 632
ROWS_SMALL = 624

MM_BLK = 400
COMB_BLK = 2000


def _mm_body(h_ref, w_ref, o_ref):
    o_ref[0] = jnp.dot(h_ref[...], w_ref[0],
                       preferred_element_type=jnp.float32)


def _compute_table(h_bf, wbd_bf):
    return pl.pallas_call(
        _mm_body,
        grid=(N_NODES // MM_BLK, NUM_RELS),
        in_specs=[
            pl.BlockSpec((MM_BLK, IN_FEAT), lambda i, r: (i, 0)),
            pl.BlockSpec((1, IN_FEAT, OUT_FEAT), lambda i, r: (r, 0, 0)),
        ],
        out_specs=pl.BlockSpec((1, MM_BLK, OUT_FEAT), lambda i, r: (r, i, 0)),
        out_shape=jax.ShapeDtypeStruct((NUM_RELS, N_NODES, OUT_FEAT),
                                       jnp.float32),
    )(h_bf, wbd_bf)


def _sc_scatter(src, ety, dst, nrm, table, zeros):
    mesh = plsc.VectorSubcoreMesh(core_axis_name="c", subcore_axis_name="s",
                                  num_cores=NC, num_subcores=NS)

    @functools.partial(
        pl.kernel,
        out_type=jax.ShapeDtypeStruct((NC, N_NODES, OUT_FEAT), jnp.float32),
        mesh=mesh,
        scratch_types=[
            pltpu.VMEM((CHUNK,), jnp.int32),
            pltpu.VMEM((CHUNK,), jnp.int32),
            pltpu.VMEM((CHUNK,), jnp.int32),
            pltpu.VMEM((CHUNK,), jnp.float32),
            pltpu.VMEM((CHUNK,), jnp.int32),
            pltpu.VMEM((CHUNK, OUT_FEAT), jnp.float32),
            pltpu.VMEM_SHARED((N_NODES, OUT_FEAT), jnp.float32),
            pltpu.SemaphoreType.DMA,
        ],
    )
    def k(src_hbm, ety_hbm, dst_hbm, nrm_hbm, t_hbm, zeros_hbm, out_hbm,
          src_v, ety_v, dst_v, nrm_v, idx_v, msg_v, acc_sh, sem):
        c = lax.axis_index("c")
        s = lax.axis_index("s")
        wid = s * NC + c
        base = wid * EDGES_PER_TILE
        row0 = pl.multiple_of(
            s * ROWS_SMALL + 8 * jnp.minimum(s, 2), 8)

        @pl.when(s < 2)
        def _():
            pltpu.sync_copy(zeros_hbm.at[pl.ds(row0, ROWS_BIG)],
                            acc_sh.at[pl.ds(row0, ROWS_BIG)])

        @pl.when(s >= 2)
        def _():
            pltpu.sync_copy(zeros_hbm.at[pl.ds(row0, ROWS_SMALL)],
                            acc_sh.at[pl.ds(row0, ROWS_SMALL)])
        plsc.subcore_barrier()

        def _chunk(j, carry):
            eb = base + j * CHUNK
            pltpu.sync_copy(src_hbm.at[pl.ds(eb, CHUNK)], src_v)
            pltpu.sync_copy(ety_hbm.at[pl.ds(eb, CHUNK)], ety_v)
            pltpu.sync_copy(dst_hbm.at[pl.ds(eb, CHUNK)], dst_v)
            pltpu.sync_copy(nrm_hbm.at[pl.ds(eb, CHUNK)], nrm_v)
            for v in range(CHUNK // 16):
                sl = pl.ds(v * 16, 16)
                idx_v[sl] = ety_v[sl] * N_NODES + src_v[sl]
            pltpu.async_copy(t_hbm.at[idx_v], msg_v, sem).wait()

            def _scale(g, cy):
                nv = nrm_v[pl.ds(g * 16, 16)]
                for l in range(16):
                    nsc = nv[l]
                    e = g * 16 + l
                    for b in range(OUT_FEAT // 16):
                        sl = pl.ds(b * 16, 16)
                        msg_v[e, sl] = msg_v[e, sl] * nsc
                return cy
            lax.fori_loop(0, CHUNK // 16, _scale, 0)
            pltpu.sync_copy(msg_v, acc_sh.at[dst_v], add=True)
            return carry
        lax.fori_loop(0, N_CHUNKS, _chunk, 0)
        plsc.subcore_barrier()

        @pl.when(s < 2)
        def _():
            pltpu.sync_copy(acc_sh.at[pl.ds(row0, ROWS_BIG)],
                            out_hbm.at[c, pl.ds(row0, ROWS_BIG)])

        @pl.when(s >= 2)
        def _():
            pltpu.sync_copy(acc_sh.at[pl.ds(row0, ROWS_SMALL)],
                            out_hbm.at[c, pl.ds(row0, ROWS_SMALL)])

    return k(src, ety, dst, nrm, table, zeros)


def _comb_body(p_ref, b_ref, o_ref):
    o_ref[...] = p_ref[0] + p_ref[1] + b_ref[...]


def _combine(partial, bias2d):
    return pl.pallas_call(
        _comb_body,
        grid=(N_NODES // COMB_BLK,),
        in_specs=[
            pl.BlockSpec((NC, COMB_BLK, OUT_FEAT), lambda i: (0, i, 0)),
            pl.BlockSpec((1, OUT_FEAT), lambda i: (0, 0)),
        ],
        out_specs=pl.BlockSpec((COMB_BLK, OUT_FEAT), lambda i: (i, 0)),
        out_shape=jax.ShapeDtypeStruct((N_NODES, OUT_FEAT), jnp.float32),
    )(partial, bias2d)


def kernel(h, edge_index, etype, norm, weight, h_bias):
    h = h.astype(jnp.float32)
    src = edge_index[0].astype(jnp.int32)
    dst = edge_index[1].astype(jnp.int32)
    ety = etype.astype(jnp.int32)
    nrm = norm.reshape(-1).astype(jnp.float32)

    w4 = weight.reshape(NUM_RELS, NUM_BASES, SI, SO)
    wbd = jnp.zeros((NUM_RELS, NUM_BASES, SI, NUM_BASES, SO), weight.dtype)
    for b in range(NUM_BASES):
        wbd = wbd.at[:, b, :, b, :].set(w4[:, b])
    wbd = wbd.reshape(NUM_RELS, IN_FEAT, OUT_FEAT)

    t = _compute_table(h.astype(jnp.bfloat16), wbd.astype(jnp.bfloat16))
    t2 = t.reshape(NUM_RELS * N_NODES, OUT_FEAT)
    zeros = jnp.zeros((N_NODES, OUT_FEAT), jnp.float32)
    part = _sc_scatter(src, ety, dst, nrm, t2, zeros)
    return _combine(part, h_bias.reshape(1, OUT_FEAT))


# speedup vs baseline: 24.4490x; 1.4121x over previous
import functools

import jax
import jax.numpy as jnp
from jax import lax
from jax.experimental import pallas as pl
from jax.experimental.pallas import tpu as pltpu
from jax.experimental.pallas import tpu_sc as plsc

N_NODES = 10000
N_EDGES = 320000
IN_FEAT = 128
OUT_FEAT = 128
NUM_RELS = 64
NUM_BASES = 8
SI = IN_FEAT // NUM_BASES
SO = OUT_FEAT // NUM_BASES

NC = 2
NS = 16
NW = NC * NS

EDGES_PER_TILE = N_EDGES // NW
CHUNK = 64
SG = 16
N_GROUPS = 10
EDGES_PAD = N_GROUPS * SG * CHUNK
ROWS_BIG = 632
ROWS_SMALL = 624

MM_BLK = 400
COMB_BLK = 2000


def _mm_body(h_ref, w_ref, o_ref):
    o_ref[0] = jnp.dot(h_ref[...], w_ref[0],
                       preferred_element_type=jnp.float32)


def _compute_table(h_bf, wbd_bf):
    return pl.pallas_call(
        _mm_body,
        grid=(NUM_RELS,),
        in_specs=[
            pl.BlockSpec((N_NODES, IN_FEAT), lambda r: (0, 0)),
            pl.BlockSpec((1, IN_FEAT, OUT_FEAT), lambda r: (r, 0, 0)),
        ],
        out_specs=pl.BlockSpec((1, N_NODES, OUT_FEAT), lambda r: (r, 0, 0)),
        out_shape=jax.ShapeDtypeStruct((NUM_RELS, N_NODES, OUT_FEAT),
                                       jnp.float32),
    )(h_bf, wbd_bf)


def _sc_scatter(src3, ety3, dst3, nrm3, table, zeros):
    mesh = plsc.VectorSubcoreMesh(core_axis_name="c", subcore_axis_name="s",
                                  num_cores=NC, num_subcores=NS)

    @functools.partial(
        pl.kernel,
        out_type=jax.ShapeDtypeStruct((NC, N_NODES, OUT_FEAT), jnp.float32),
        mesh=mesh,
        scratch_types=[
            pltpu.VMEM((SG, CHUNK), jnp.int32),
            pltpu.VMEM((SG, CHUNK), jnp.int32),
            pltpu.VMEM((SG, CHUNK), jnp.int32),
            pltpu.VMEM((SG, CHUNK), jnp.float32),
            pltpu.VMEM((2, CHUNK, OUT_FEAT), jnp.float32),
            pltpu.VMEM_SHARED((N_NODES, OUT_FEAT), jnp.float32),
            pltpu.SemaphoreType.DMA((2,)),
            pltpu.SemaphoreType.DMA((2,)),
        ],
    )
    def k(src_hbm, ety_hbm, dst_hbm, nrm_hbm, t_hbm, zeros_hbm, out_hbm,
          src_v, idx_v, dst_v, nrm_v, msg_v, acc_sh, gsem, ssem):
        c = lax.axis_index("c")
        s = lax.axis_index("s")
        wid = s * NC + c
        row0 = pl.multiple_of(
            s * ROWS_SMALL + 8 * jnp.minimum(s, 2), 8)

        @pl.when(s < 2)
        def _():
            pltpu.sync_copy(zeros_hbm.at[pl.ds(row0, ROWS_BIG)],
                            acc_sh.at[pl.ds(row0, ROWS_BIG)])

        @pl.when(s >= 2)
        def _():
            pltpu.sync_copy(zeros_hbm.at[pl.ds(row0, ROWS_SMALL)],
                            acc_sh.at[pl.ds(row0, ROWS_SMALL)])

        plsc.subcore_barrier()

        def _gather(cc, b):
            return pltpu.async_copy(t_hbm.at[idx_v.at[cc]], msg_v.at[b],
                                    gsem.at[b])

        def _wait_scatter(cc, b):
            pltpu.make_async_copy(msg_v.at[b], acc_sh.at[dst_v.at[cc]],
                                  ssem.at[b]).wait()

        def _group(g, cy):
            c0 = pl.multiple_of(g * SG, SG)
            pltpu.sync_copy(src_hbm.at[wid, pl.ds(c0, SG)], src_v)
            pltpu.sync_copy(ety_hbm.at[wid, pl.ds(c0, SG)], idx_v)
            pltpu.sync_copy(dst_hbm.at[wid, pl.ds(c0, SG)], dst_v)
            pltpu.sync_copy(nrm_hbm.at[wid, pl.ds(c0, SG)], nrm_v)

            def _cidx(j, cy2):
                for v in range(CHUNK // 16):
                    sl = pl.ds(v * 16, 16)
                    idx_v[j, sl] = idx_v[j, sl] * N_NODES + src_v[j, sl]
                return cy2
            lax.fori_loop(0, SG, _cidx, 0)

            _gather(0, 0)

            def _step(cc, cy2):
                b = lax.rem(cc, 2)
                pltpu.make_async_copy(t_hbm.at[idx_v.at[cc]], msg_v.at[b],
                                      gsem.at[b]).wait()

                def _scale(gg, cy3):
                    nv = nrm_v[cc, pl.ds(gg * 16, 16)]
                    for l in range(16):
                        nsc = nv[l]
                        e = gg * 16 + l
                        for bb in range(OUT_FEAT // 16):
                            sl = pl.ds(bb * 16, 16)
                            msg_v[b, e, sl] = msg_v[b, e, sl] * nsc
                    return cy3
                lax.fori_loop(0, CHUNK // 16, _scale, 0)
                pltpu.async_copy(msg_v.at[b], acc_sh.at[dst_v.at[cc]],
                                 ssem.at[b], add=True)

                @pl.when(cc <= SG - 2)
                def _():
                    @pl.when(cc >= 1)
                    def _():
                        _wait_scatter(cc - 1, 1 - b)
                    _gather(cc + 1, 1 - b)
                return cy2
            lax.fori_loop(0, SG, _step, 0)

            _wait_scatter(SG - 2, lax.rem(SG - 2, 2))
            _wait_scatter(SG - 1, lax.rem(SG - 1, 2))
            return cy
        lax.fori_loop(0, N_GROUPS, _group, 0)
        plsc.subcore_barrier()

        @pl.when(s < 2)
        def _():
            pltpu.sync_copy(acc_sh.at[pl.ds(row0, ROWS_BIG)],
                            out_hbm.at[c, pl.ds(row0, ROWS_BIG)])

        @pl.when(s >= 2)
        def _():
            pltpu.sync_copy(acc_sh.at[pl.ds(row0, ROWS_SMALL)],
                            out_hbm.at[c, pl.ds(row0, ROWS_SMALL)])

    return k(src3, ety3, dst3, nrm3, table, zeros)


def _comb_body(p_ref, b_ref, o_ref):
    o_ref[...] = p_ref[0] + p_ref[1] + b_ref[...]


def _combine(partial, bias2d):
    return pl.pallas_call(
        _comb_body,
        grid=(N_NODES // COMB_BLK,),
        in_specs=[
            pl.BlockSpec((NC, COMB_BLK, OUT_FEAT), lambda i: (0, i, 0)),
            pl.BlockSpec((1, OUT_FEAT), lambda i: (0, 0)),
        ],
        out_specs=pl.BlockSpec((COMB_BLK, OUT_FEAT), lambda i: (i, 0)),
        out_shape=jax.ShapeDtypeStruct((N_NODES, OUT_FEAT), jnp.float32),
    )(partial, bias2d)


def kernel(h, edge_index, etype, norm, weight, h_bias):
    h = h.astype(jnp.float32)
    src = edge_index[0].astype(jnp.int32)
    dst = edge_index[1].astype(jnp.int32)
    ety = etype.astype(jnp.int32)
    nrm = norm.reshape(-1).astype(jnp.float32)

    w4 = weight.reshape(NUM_RELS, NUM_BASES, SI, SO)
    wbd = jnp.zeros((NUM_RELS, NUM_BASES, SI, NUM_BASES, SO), weight.dtype)
    for b in range(NUM_BASES):
        wbd = wbd.at[:, b, :, b, :].set(w4[:, b])
    wbd = wbd.reshape(NUM_RELS, IN_FEAT, OUT_FEAT)

    t = _compute_table(h.astype(jnp.bfloat16), wbd.astype(jnp.bfloat16))
    t2 = t.reshape(NUM_RELS * N_NODES, OUT_FEAT)
    zeros = jnp.zeros((N_NODES, OUT_FEAT), jnp.float32)

    pad = ((0, 0), (0, EDGES_PAD - EDGES_PER_TILE))
    shp = (NW, N_GROUPS * SG, CHUNK)
    srcp = jnp.pad(src.reshape(NW, EDGES_PER_TILE), pad).reshape(shp)
    etyp = jnp.pad(ety.reshape(NW, EDGES_PER_TILE), pad).reshape(shp)
    dstp = jnp.pad(dst.reshape(NW, EDGES_PER_TILE), pad).reshape(shp)
    nrmp = jnp.pad(nrm.reshape(NW, EDGES_PER_TILE), pad).reshape(shp)
    part = _sc_scatter(srcp, etyp, dstp, nrmp, t2, zeros)
    return _combine(part, h_bias.reshape(1, OUT_FEAT))
